# baseline (speedup 1.0000x reference)
import jax
import jax.numpy as jnp
from jax import lax
from jax.experimental import pallas as pl
from jax.experimental.pallas import tpu as pltpu
from jax.experimental.pallas import tpu_sc as plsc

N = 10000
E = 160000
NPAD = 10240
EPAD = 163840
CHUNK = 128
ROWS = EPAD // CHUNK
NC, NS = 2, 16
NW = NC * NS
WROWS = ROWS // NW
NSUB = NPAD // NS
F = 16

_mesh = plsc.VectorSubcoreMesh(core_axis_name="c", subcore_axis_name="s")
_sc_params = pltpu.CompilerParams(needs_layout_passes=False,
                                  use_tc_tiling_on_sc=False)


def _zero_fill(ref, nrows):
    def body(i, _):
        ref[i] = jnp.zeros((16,), jnp.float32)
        return 0
    lax.fori_loop(0, nrows, body, 0)


def _sc_agg(src_hbm, dst_hbm, ew_hbm, h_hbm, agg_out,
            src_v, dst_v, ew_v, rows_v, zn_v, gsem, ssem, acc_sh):
    c = lax.axis_index("c")
    s = lax.axis_index("s")
    w = c * NS + s
    nsl = pl.ds(s * NSUB, NSUB)

    _zero_fill(zn_v, NSUB)
    pltpu.sync_copy(zn_v, acc_sh.at[nsl])

    wsl = pl.ds(w * WROWS, WROWS)
    pltpu.sync_copy(src_hbm.at[wsl], src_v)
    pltpu.sync_copy(dst_hbm.at[wsl], dst_v)
    pltpu.sync_copy(ew_hbm.at[wsl], ew_v)
    plsc.subcore_barrier()

    pltpu.async_copy(h_hbm.at[src_v.at[0]], rows_v.at[0], gsem)

    def outer(g, _):
        for b in range(2):
            ci = g * 2 + b
            pltpu.make_async_copy(
                h_hbm.at[src_v.at[0]], rows_v.at[b], gsem).wait()

            @pl.when(ci >= 1)
            def _():
                pltpu.make_async_copy(
                    rows_v.at[1 - b], acc_sh.at[dst_v.at[0]], ssem).wait()

            @pl.when(ci + 1 < WROWS)
            def _():
                pltpu.async_copy(
                    h_hbm.at[src_v.at[ci + 1]], rows_v.at[1 - b], gsem)

            for k in range(CHUNK // 16):
                base = k * 16
                ewv = ew_v[ci, pl.ds(base, 16)]
                for u in range(16):
                    j = base + u
                    rows_v[b, j] = rows_v[b, j] * ewv[u]

            pltpu.async_copy(rows_v.at[b], acc_sh.at[dst_v.at[ci]], ssem,
                             add=True)
        return 0
    lax.fori_loop(0, WROWS // 2, outer, 0)
    pltpu.make_async_copy(rows_v.at[1], acc_sh.at[dst_v.at[0]], ssem).wait()
    plsc.subcore_barrier()
    pltpu.sync_copy(acc_sh.at[nsl], agg_out.at[c, nsl])


_agg_call = pl.kernel(
    _sc_agg,
    out_type=jax.ShapeDtypeStruct((NC, NPAD, F), jnp.float32),
    mesh=_mesh,
    scratch_types=[
        pltpu.VMEM((WROWS, CHUNK), jnp.int32),
        pltpu.VMEM((WROWS, CHUNK), jnp.int32),
        pltpu.VMEM((WROWS, CHUNK), jnp.float32),
        pltpu.VMEM((2, CHUNK, F), jnp.float32),
        pltpu.VMEM((NSUB, F), jnp.float32),
        pltpu.SemaphoreType.DMA,
        pltpu.SemaphoreType.DMA,
        pltpu.VMEM_SHARED((NPAD, F), jnp.float32),
    ],
    compiler_params=_sc_params,
)


def _tc_first(x_ref, w_ref, degp_ref, t_ref, dis_ref):
    dis = lax.rsqrt(degp_ref[0] + degp_ref[1] + 1.0)
    dis_ref[...] = dis
    h0 = jnp.dot(x_ref[...], w_ref[...], preferred_element_type=jnp.float32)
    t_ref[...] = dis[:N] * h0


def _tc_mid(agg_ref, t_ref, dis_ref, b_ref, w_ref, o_ref):
    dis = dis_ref[...][:N]
    h = jax.nn.relu(dis * (agg_ref[0, :N] + agg_ref[1, :N] + t_ref[...])
                    + b_ref[...])
    o_ref[...] = dis * jnp.dot(h, w_ref[...],
                               preferred_element_type=jnp.float32)


def _tc_final(agg_ref, t_ref, dis_ref, bmu_ref, blv_ref, mu_ref, lv_ref):
    dis = dis_ref[...][:N]
    out = dis * (agg_ref[0, :N] + agg_ref[1, :N] + t_ref[...])
    mu_ref[...] = out[:, 0:2] + bmu_ref[...]
    lv_ref[...] = out[:, 2:4] + blv_ref[...]


def _tc(body, out_shape, *args):
    return pl.pallas_call(body, out_shape=out_shape)(*args)


@jax.jit
def kernel(x, edge_index, edge_weight, W1, b1, W2, b2, Wmu, bmu, Wlv, blv):
    f32 = jnp.float32
    src = edge_index[0].astype(jnp.int32)
    dst = edge_index[1].astype(jnp.int32)
    ew = edge_weight.astype(f32)

    npad_e = EPAD - E
    fill = jnp.arange(npad_e, dtype=jnp.int32)
    src_p = jnp.concatenate([src, fill % N]).reshape(ROWS, CHUNK)
    dst_p = jnp.concatenate([dst, N + fill % (NPAD - N)]).reshape(ROWS, CHUNK)
    ew_p = jnp.concatenate([ew, jnp.zeros((npad_e,), f32)]).reshape(ROWS, CHUNK)

    W2p = jnp.pad(W2, ((0, 0), (0, F - W2.shape[1])))
    Wout = jnp.concatenate([Wmu, Wlv], axis=1)
    Woutp = jnp.pad(Wout, ((0, F - Wout.shape[0]), (0, F - Wout.shape[1])))
    b1r = b1.reshape(1, F)
    b2r = jnp.pad(b2, (0, F - b2.shape[0])).reshape(1, F)
    bmur = bmu.reshape(1, 2)
    blvr = blv.reshape(1, 2)

    degp = _agg_call(src_p, dst_p, ew_p, jnp.ones((N, F), f32))
    t0, dis = _tc(_tc_first,
                  (jax.ShapeDtypeStruct((N, F), f32),
                   jax.ShapeDtypeStruct((NPAD, F), f32)),
                  x, W1, degp)

    agg1 = _agg_call(src_p, dst_p, ew_p, t0)
    t1 = _tc(_tc_mid, jax.ShapeDtypeStruct((N, F), f32),
             agg1, t0, dis, b1r, W2p)
    agg2 = _agg_call(src_p, dst_p, ew_p, t1)
    t2 = _tc(_tc_mid, jax.ShapeDtypeStruct((N, F), f32),
             agg2, t1, dis, b2r, Woutp)
    agg3 = _agg_call(src_p, dst_p, ew_p, t2)
    mu, lv = _tc(_tc_final,
                 (jax.ShapeDtypeStruct((N, 2), f32),
                  jax.ShapeDtypeStruct((N, 2), f32)),
                 agg3, t2, dis, bmur, blvr)
    return (mu, lv)


# speedup vs baseline: 1.3071x; 1.3071x over previous
import jax
import jax.numpy as jnp
from jax import lax
from jax.experimental import pallas as pl
from jax.experimental.pallas import tpu as pltpu
from jax.experimental.pallas import tpu_sc as plsc

N = 10000
E = 160000
NPAD = 10240
EPAD = 163840
CHUNK = 256
ROWS = EPAD // CHUNK
NC, NS = 2, 16
NW = NC * NS
WROWS = ROWS // NW
NSUB = NPAD // NS
F = 16

_mesh = plsc.VectorSubcoreMesh(core_axis_name="c", subcore_axis_name="s")
_sc_params = pltpu.CompilerParams(needs_layout_passes=False,
                                  use_tc_tiling_on_sc=False)


def _zero_fill(ref, nrows):
    def body(i, _):
        ref[i] = jnp.zeros((16,), jnp.float32)
        return 0
    lax.fori_loop(0, nrows, body, 0)


def _sc_deg(dst_hbm, ew_hbm, deg_out, idx_v, val_v, z_v, rep_v, sem, deg_sh):
    c = lax.axis_index("c")
    s = lax.axis_index("s")
    w = c * NS + s
    nsl = pl.ds(s * NSUB, NSUB)

    def zbody(k, _):
        z_v[pl.ds(k * 16, 16)] = jnp.zeros((16,), jnp.float32)
        return 0
    lax.fori_loop(0, NSUB // 16, zbody, 0)
    pltpu.sync_copy(z_v, deg_sh.at[nsl])

    wsl = pl.ds(w * WROWS, WROWS)
    pltpu.sync_copy(dst_hbm.at[wsl], idx_v)
    pltpu.sync_copy(ew_hbm.at[wsl], val_v)
    plsc.subcore_barrier()

    def fire(r, _):
        pltpu.async_copy(val_v.at[r], deg_sh.at[idx_v.at[r]], sem, add=True)
        return 0
    lax.fori_loop(0, WROWS, fire, 0)

    def drain(r, _):
        pltpu.make_async_copy(val_v.at[0], deg_sh.at[idx_v.at[0]], sem).wait()
        return 0
    lax.fori_loop(0, WROWS, drain, 0)
    plsc.subcore_barrier()

    pltpu.sync_copy(deg_sh.at[nsl], z_v)

    def rep(k, _):
        dv = z_v[pl.ds(k * 16, 16)]
        for u in range(16):
            rep_v[k * 16 + u] = jnp.full((16,), dv[u], jnp.float32)
        return 0
    lax.fori_loop(0, NSUB // 16, rep, 0)
    pltpu.sync_copy(rep_v, deg_out.at[c, nsl])


def _sc_agg(src_hbm, dst_hbm, ew_hbm, h_hbm, agg_out,
            src_v, dst_v, ew_v, rows_v, zn_v, gsem, ssem, acc_sh):
    c = lax.axis_index("c")
    s = lax.axis_index("s")
    w = c * NS + s
    nsl = pl.ds(s * NSUB, NSUB)

    _zero_fill(zn_v, NSUB)
    pltpu.sync_copy(zn_v, acc_sh.at[nsl])

    wsl = pl.ds(w * WROWS, WROWS)
    pltpu.sync_copy(src_hbm.at[wsl], src_v)
    pltpu.sync_copy(dst_hbm.at[wsl], dst_v)
    pltpu.sync_copy(ew_hbm.at[wsl], ew_v)
    plsc.subcore_barrier()

    pltpu.async_copy(h_hbm.at[src_v.at[0]], rows_v.at[0], gsem)

    def outer(g, _):
        for b in range(2):
            ci = g * 2 + b
            pltpu.make_async_copy(
                h_hbm.at[src_v.at[0]], rows_v.at[b], gsem).wait()

            @pl.when(ci >= 1)
            def _():
                pltpu.make_async_copy(
                    rows_v.at[1 - b], acc_sh.at[dst_v.at[0]], ssem).wait()

            @pl.when(ci + 1 < WROWS)
            def _():
                pltpu.async_copy(
                    h_hbm.at[src_v.at[ci + 1]], rows_v.at[1 - b], gsem)

            for k in range(CHUNK // 16):
                base = k * 16
                ewv = ew_v[ci, pl.ds(base, 16)]
                for u in range(16):
                    j = base + u
                    rows_v[b, j] = rows_v[b, j] * ewv[u]

            pltpu.async_copy(rows_v.at[b], acc_sh.at[dst_v.at[ci]], ssem,
                             add=True)
        return 0
    lax.fori_loop(0, WROWS // 2, outer, 0)
    pltpu.make_async_copy(rows_v.at[1], acc_sh.at[dst_v.at[0]], ssem).wait()
    plsc.subcore_barrier()
    pltpu.sync_copy(acc_sh.at[nsl], agg_out.at[c, nsl])


_deg_call = pl.kernel(
    _sc_deg,
    out_type=jax.ShapeDtypeStruct((NC, NPAD, F), jnp.float32),
    mesh=_mesh,
    scratch_types=[
        pltpu.VMEM((WROWS, CHUNK), jnp.int32),
        pltpu.VMEM((WROWS, CHUNK), jnp.float32),
        pltpu.VMEM((NSUB,), jnp.float32),
        pltpu.VMEM((NSUB, F), jnp.float32),
        pltpu.SemaphoreType.DMA,
        pltpu.VMEM_SHARED((NPAD,), jnp.float32),
    ],
    compiler_params=_sc_params,
)

_agg_call = pl.kernel(
    _sc_agg,
    out_type=jax.ShapeDtypeStruct((NC, NPAD, F), jnp.float32),
    mesh=_mesh,
    scratch_types=[
        pltpu.VMEM((WROWS, CHUNK), jnp.int32),
        pltpu.VMEM((WROWS, CHUNK), jnp.int32),
        pltpu.VMEM((WROWS, CHUNK), jnp.float32),
        pltpu.VMEM((2, CHUNK, F), jnp.float32),
        pltpu.VMEM((NSUB, F), jnp.float32),
        pltpu.SemaphoreType.DMA,
        pltpu.SemaphoreType.DMA,
        pltpu.VMEM_SHARED((NPAD, F), jnp.float32),
    ],
    compiler_params=_sc_params,
)


def _tc_first(x_ref, w_ref, degp_ref, t_ref, dis_ref):
    dis = lax.rsqrt(degp_ref[0] + degp_ref[1] + 1.0)
    dis_ref[...] = dis
    h0 = jnp.dot(x_ref[...], w_ref[...], preferred_element_type=jnp.float32)
    t_ref[...] = dis[:N] * h0


def _tc_mid(agg_ref, t_ref, dis_ref, b_ref, w_ref, o_ref):
    dis = dis_ref[...][:N]
    h = jax.nn.relu(dis * (agg_ref[0, :N] + agg_ref[1, :N] + t_ref[...])
                    + b_ref[...])
    o_ref[...] = dis * jnp.dot(h, w_ref[...],
                               preferred_element_type=jnp.float32)


def _tc_final(agg_ref, t_ref, dis_ref, bmu_ref, blv_ref, mu_ref, lv_ref):
    dis = dis_ref[...][:N]
    out = dis * (agg_ref[0, :N] + agg_ref[1, :N] + t_ref[...])
    mu_ref[...] = out[:, 0:2] + bmu_ref[...]
    lv_ref[...] = out[:, 2:4] + blv_ref[...]


def _tc(body, out_shape, *args):
    return pl.pallas_call(body, out_shape=out_shape)(*args)


@jax.jit
def kernel(x, edge_index, edge_weight, W1, b1, W2, b2, Wmu, bmu, Wlv, blv):
    f32 = jnp.float32
    src = edge_index[0].astype(jnp.int32)
    dst = edge_index[1].astype(jnp.int32)
    ew = edge_weight.astype(f32)

    npad_e = EPAD - E
    fill = jnp.arange(npad_e, dtype=jnp.int32)
    src_p = jnp.concatenate([src, fill % N]).reshape(ROWS, CHUNK)
    dst_p = jnp.concatenate([dst, N + fill % (NPAD - N)]).reshape(ROWS, CHUNK)
    ew_p = jnp.concatenate([ew, jnp.zeros((npad_e,), f32)]).reshape(ROWS, CHUNK)

    W2p = jnp.pad(W2, ((0, 0), (0, F - W2.shape[1])))
    Wout = jnp.concatenate([Wmu, Wlv], axis=1)
    Woutp = jnp.pad(Wout, ((0, F - Wout.shape[0]), (0, F - Wout.shape[1])))
    b1r = b1.reshape(1, F)
    b2r = jnp.pad(b2, (0, F - b2.shape[0])).reshape(1, F)
    bmur = bmu.reshape(1, 2)
    blvr = blv.reshape(1, 2)

    degp = _deg_call(dst_p, ew_p)
    t0, dis = _tc(_tc_first,
                  (jax.ShapeDtypeStruct((N, F), f32),
                   jax.ShapeDtypeStruct((NPAD, F), f32)),
                  x, W1, degp)

    agg1 = _agg_call(src_p, dst_p, ew_p, t0)
    t1 = _tc(_tc_mid, jax.ShapeDtypeStruct((N, F), f32),
             agg1, t0, dis, b1r, W2p)
    agg2 = _agg_call(src_p, dst_p, ew_p, t1)
    t2 = _tc(_tc_mid, jax.ShapeDtypeStruct((N, F), f32),
             agg2, t1, dis, b2r, Woutp)
    agg3 = _agg_call(src_p, dst_p, ew_p, t2)
    mu, lv = _tc(_tc_final,
                 (jax.ShapeDtypeStruct((N, 2), f32),
                  jax.ShapeDtypeStruct((N, 2), f32)),
                 agg3, t2, dis, bmur, blvr)
    return (mu, lv)


# speedup vs baseline: 1.3180x; 1.0084x over previous
import jax
import jax.numpy as jnp
from jax import lax
from jax.experimental import pallas as pl
from jax.experimental.pallas import tpu as pltpu
from jax.experimental.pallas import tpu_sc as plsc

N = 10000
E = 160000
NPAD = 10240
EPAD = 163840
CHUNK = 256
ROWS = EPAD // CHUNK
NC, NS = 2, 16
NW = NC * NS
WROWS = ROWS // NW
NSUB = NPAD // NS
F = 16

_mesh = plsc.VectorSubcoreMesh(core_axis_name="c", subcore_axis_name="s")
_sc_params = pltpu.CompilerParams(needs_layout_passes=False,
                                  use_tc_tiling_on_sc=False)


def _zero_fill(ref, nrows):
    def body(i, _):
        ref[i] = jnp.zeros((16,), jnp.float32)
        return 0
    lax.fori_loop(0, nrows, body, 0)


def _sc_deg(dst_hbm, ew_hbm, deg_out, idx_v, val_v, z_v, rep_v, sem, deg_sh):
    c = lax.axis_index("c")
    s = lax.axis_index("s")
    w = c * NS + s
    nsl = pl.ds(s * NSUB, NSUB)

    def zbody(k, _):
        z_v[pl.ds(k * 16, 16)] = jnp.zeros((16,), jnp.float32)
        return 0
    lax.fori_loop(0, NSUB // 16, zbody, 0)
    pltpu.sync_copy(z_v, deg_sh.at[nsl])

    wsl = pl.ds(w * WROWS, WROWS)
    pltpu.sync_copy(dst_hbm.at[wsl], idx_v)
    pltpu.sync_copy(ew_hbm.at[wsl], val_v)
    plsc.subcore_barrier()

    def fire(r, _):
        pltpu.async_copy(val_v.at[r], deg_sh.at[idx_v.at[r]], sem, add=True)
        return 0
    lax.fori_loop(0, WROWS, fire, 0)

    def drain(r, _):
        pltpu.make_async_copy(val_v.at[0], deg_sh.at[idx_v.at[0]], sem).wait()
        return 0
    lax.fori_loop(0, WROWS, drain, 0)
    plsc.subcore_barrier()

    pltpu.sync_copy(deg_sh.at[nsl], z_v)

    def rep(k, _):
        dv = z_v[pl.ds(k * 16, 16)]
        for u in range(16):
            rep_v[k * 16 + u] = jnp.full((16,), dv[u], jnp.float32)
        return 0
    lax.fori_loop(0, NSUB // 16, rep, 0)
    pltpu.sync_copy(rep_v, deg_out.at[c, nsl])


def _sc_agg(src_hbm, dst_hbm, ew_hbm, h_hbm, agg_out,
            src_v, dst_v, ew_v, rows_v, zn_v, gsem, ssem, acc_sh):
    c = lax.axis_index("c")
    s = lax.axis_index("s")
    w = c * NS + s
    nsl = pl.ds(s * NSUB, NSUB)

    _zero_fill(zn_v, NSUB)
    pltpu.sync_copy(zn_v, acc_sh.at[nsl])

    wsl = pl.ds(w * WROWS, WROWS)
    pltpu.sync_copy(src_hbm.at[wsl], src_v)
    pltpu.sync_copy(dst_hbm.at[wsl], dst_v)
    pltpu.sync_copy(ew_hbm.at[wsl], ew_v)
    plsc.subcore_barrier()

    pltpu.async_copy(h_hbm.at[src_v.at[0]], rows_v.at[0], gsem)
    pltpu.async_copy(h_hbm.at[src_v.at[1]], rows_v.at[1], gsem)

    def outer(g, _):
        for b in range(4):
            ci = g * 4 + b
            pltpu.make_async_copy(
                h_hbm.at[src_v.at[0]], rows_v.at[b], gsem).wait()

            @pl.when(ci >= 2)
            def _():
                pltpu.make_async_copy(
                    rows_v.at[(b + 2) % 4], acc_sh.at[dst_v.at[0]],
                    ssem).wait()

            @pl.when(ci + 2 < WROWS)
            def _():
                pltpu.async_copy(
                    h_hbm.at[src_v.at[ci + 2]], rows_v.at[(b + 2) % 4], gsem)

            for k in range(CHUNK // 16):
                base = k * 16
                ewv = ew_v[ci, pl.ds(base, 16)]
                for u in range(16):
                    j = base + u
                    rows_v[b, j] = rows_v[b, j] * ewv[u]

            pltpu.async_copy(rows_v.at[b], acc_sh.at[dst_v.at[ci]], ssem,
                             add=True)
        return 0
    lax.fori_loop(0, WROWS // 4, outer, 0)
    pltpu.make_async_copy(rows_v.at[0], acc_sh.at[dst_v.at[0]], ssem).wait()
    pltpu.make_async_copy(rows_v.at[1], acc_sh.at[dst_v.at[0]], ssem).wait()
    plsc.subcore_barrier()
    pltpu.sync_copy(acc_sh.at[nsl], agg_out.at[c, nsl])


_deg_call = pl.kernel(
    _sc_deg,
    out_type=jax.ShapeDtypeStruct((NC, NPAD, F), jnp.float32),
    mesh=_mesh,
    scratch_types=[
        pltpu.VMEM((WROWS, CHUNK), jnp.int32),
        pltpu.VMEM((WROWS, CHUNK), jnp.float32),
        pltpu.VMEM((NSUB,), jnp.float32),
        pltpu.VMEM((NSUB, F), jnp.float32),
        pltpu.SemaphoreType.DMA,
        pltpu.VMEM_SHARED((NPAD,), jnp.float32),
    ],
    compiler_params=_sc_params,
)

_agg_call = pl.kernel(
    _sc_agg,
    out_type=jax.ShapeDtypeStruct((NC, NPAD, F), jnp.float32),
    mesh=_mesh,
    scratch_types=[
        pltpu.VMEM((WROWS, CHUNK), jnp.int32),
        pltpu.VMEM((WROWS, CHUNK), jnp.int32),
        pltpu.VMEM((WROWS, CHUNK), jnp.float32),
        pltpu.VMEM((4, CHUNK, F), jnp.float32),
        pltpu.VMEM((NSUB, F), jnp.float32),
        pltpu.SemaphoreType.DMA,
        pltpu.SemaphoreType.DMA,
        pltpu.VMEM_SHARED((NPAD, F), jnp.float32),
    ],
    compiler_params=_sc_params,
)


def _tc_first(x_ref, w_ref, degp_ref, t_ref, dis_ref):
    dis = lax.rsqrt(degp_ref[0] + degp_ref[1] + 1.0)
    dis_ref[...] = dis
    h0 = jnp.dot(x_ref[...], w_ref[...], preferred_element_type=jnp.float32)
    t_ref[...] = dis[:N] * h0


def _tc_mid(agg_ref, t_ref, dis_ref, b_ref, w_ref, o_ref):
    dis = dis_ref[...][:N]
    h = jax.nn.relu(dis * (agg_ref[0, :N] + agg_ref[1, :N] + t_ref[...])
                    + b_ref[...])
    o_ref[...] = dis * jnp.dot(h, w_ref[...],
                               preferred_element_type=jnp.float32)


def _tc_final(agg_ref, t_ref, dis_ref, bmu_ref, blv_ref, mu_ref, lv_ref):
    dis = dis_ref[...][:N]
    out = dis * (agg_ref[0, :N] + agg_ref[1, :N] + t_ref[...])
    mu_ref[...] = out[:, 0:2] + bmu_ref[...]
    lv_ref[...] = out[:, 2:4] + blv_ref[...]


def _tc(body, out_shape, *args):
    return pl.pallas_call(body, out_shape=out_shape)(*args)


@jax.jit
def kernel(x, edge_index, edge_weight, W1, b1, W2, b2, Wmu, bmu, Wlv, blv):
    f32 = jnp.float32
    src = edge_index[0].astype(jnp.int32)
    dst = edge_index[1].astype(jnp.int32)
    ew = edge_weight.astype(f32)

    npad_e = EPAD - E
    fill = jnp.arange(npad_e, dtype=jnp.int32)
    src_p = jnp.concatenate([src, fill % N]).reshape(ROWS, CHUNK)
    dst_p = jnp.concatenate([dst, N + fill % (NPAD - N)]).reshape(ROWS, CHUNK)
    ew_p = jnp.concatenate([ew, jnp.zeros((npad_e,), f32)]).reshape(ROWS, CHUNK)

    W2p = jnp.pad(W2, ((0, 0), (0, F - W2.shape[1])))
    Wout = jnp.concatenate([Wmu, Wlv], axis=1)
    Woutp = jnp.pad(Wout, ((0, F - Wout.shape[0]), (0, F - Wout.shape[1])))
    b1r = b1.reshape(1, F)
    b2r = jnp.pad(b2, (0, F - b2.shape[0])).reshape(1, F)
    bmur = bmu.reshape(1, 2)
    blvr = blv.reshape(1, 2)

    degp = _deg_call(dst_p, ew_p)
    t0, dis = _tc(_tc_first,
                  (jax.ShapeDtypeStruct((N, F), f32),
                   jax.ShapeDtypeStruct((NPAD, F), f32)),
                  x, W1, degp)

    agg1 = _agg_call(src_p, dst_p, ew_p, t0)
    t1 = _tc(_tc_mid, jax.ShapeDtypeStruct((N, F), f32),
             agg1, t0, dis, b1r, W2p)
    agg2 = _agg_call(src_p, dst_p, ew_p, t1)
    t2 = _tc(_tc_mid, jax.ShapeDtypeStruct((N, F), f32),
             agg2, t1, dis, b2r, Woutp)
    agg3 = _agg_call(src_p, dst_p, ew_p, t2)
    mu, lv = _tc(_tc_final,
                 (jax.ShapeDtypeStruct((N, 2), f32),
                  jax.ShapeDtypeStruct((N, 2), f32)),
                 agg3, t2, dis, bmur, blvr)
    return (mu, lv)


# speedup vs baseline: 1.6356x; 1.2409x over previous
import jax
import jax.numpy as jnp
from jax import lax
from jax.experimental import pallas as pl
from jax.experimental.pallas import tpu as pltpu
from jax.experimental.pallas import tpu_sc as plsc

N = 10000
E = 160000
NPAD = 10240
EPAD = 163840
CHUNK = 256
ROWS = EPAD // CHUNK
NC, NS = 2, 16
NW = NC * NS
WROWS = ROWS // NW
NSUB = NPAD // NS
F = 16

_mesh = plsc.VectorSubcoreMesh(core_axis_name="c", subcore_axis_name="s")
_sc_params = pltpu.CompilerParams(needs_layout_passes=False,
                                  use_tc_tiling_on_sc=False)


def _zero_fill(ref, nrows):
    def body(i, _):
        ref[i] = jnp.zeros((16,), jnp.float32)
        return 0
    lax.fori_loop(0, nrows, body, 0)


def _sc_deg(dst_hbm, ew_hbm, deg_out, idx_v, val_v, z_v, rep_v, sem, deg_sh):
    c = lax.axis_index("c")
    s = lax.axis_index("s")
    w = c * NS + s
    nsl = pl.ds(s * NSUB, NSUB)

    def zbody(k, _):
        z_v[pl.ds(k * 16, 16)] = jnp.zeros((16,), jnp.float32)
        return 0
    lax.fori_loop(0, NSUB // 16, zbody, 0)
    pltpu.sync_copy(z_v, deg_sh.at[nsl])

    wsl = pl.ds(w * WROWS, WROWS)
    pltpu.sync_copy(dst_hbm.at[wsl], idx_v)
    pltpu.sync_copy(ew_hbm.at[wsl], val_v)
    plsc.subcore_barrier()

    def fire(r, _):
        pltpu.async_copy(val_v.at[r], deg_sh.at[idx_v.at[r]], sem, add=True)
        return 0
    lax.fori_loop(0, WROWS, fire, 0)

    def drain(r, _):
        pltpu.make_async_copy(val_v.at[0], deg_sh.at[idx_v.at[0]], sem).wait()
        return 0
    lax.fori_loop(0, WROWS, drain, 0)
    plsc.subcore_barrier()

    pltpu.sync_copy(deg_sh.at[nsl], z_v)

    def rep(k, _):
        dv = z_v[pl.ds(k * 16, 16)]
        for u in range(16):
            rep_v[k * 16 + u] = jnp.full((16,), dv[u], jnp.float32)
        return 0
    lax.fori_loop(0, NSUB // 16, rep, 0)
    pltpu.sync_copy(rep_v, deg_out.at[c, nsl])


def _sc_agg(src_hbm, dst_hbm, ew_hbm, h_hbm, agg_out,
            src_v, dst_v, ew_v, rows_v, zn_v, gsem, ssem, acc_sh):
    c = lax.axis_index("c")
    s = lax.axis_index("s")
    w = c * NS + s
    nsl = pl.ds(s * NSUB, NSUB)

    _zero_fill(zn_v, NSUB)
    pltpu.sync_copy(zn_v, acc_sh.at[nsl])

    wsl = pl.ds(w * WROWS, WROWS)
    pltpu.sync_copy(src_hbm.at[wsl], src_v)
    pltpu.sync_copy(dst_hbm.at[wsl], dst_v)
    pltpu.sync_copy(ew_hbm.at[wsl], ew_v)
    plsc.subcore_barrier()

    pltpu.async_copy(h_hbm.at[src_v.at[0]], rows_v.at[0], gsem)
    pltpu.async_copy(h_hbm.at[src_v.at[1]], rows_v.at[1], gsem)

    def outer(g, _):
        for b in range(4):
            ci = g * 4 + b
            pltpu.make_async_copy(
                h_hbm.at[src_v.at[0]], rows_v.at[b], gsem).wait()

            @pl.when(ci >= 2)
            def _():
                pltpu.make_async_copy(
                    rows_v.at[(b + 2) % 4], acc_sh.at[dst_v.at[0]],
                    ssem).wait()

            @pl.when(ci + 2 < WROWS)
            def _():
                pltpu.async_copy(
                    h_hbm.at[src_v.at[ci + 2]], rows_v.at[(b + 2) % 4], gsem)

            for k in range(CHUNK // 16):
                base = k * 16
                ewv = ew_v[ci, pl.ds(base, 16)]
                for u in range(16):
                    j = base + u
                    rows_v[b, j] = rows_v[b, j] * ewv[u]

            pltpu.async_copy(rows_v.at[b], acc_sh.at[dst_v.at[ci]], ssem,
                             add=True)
        return 0
    lax.fori_loop(0, WROWS // 4, outer, 0)
    pltpu.make_async_copy(rows_v.at[0], acc_sh.at[dst_v.at[0]], ssem).wait()
    pltpu.make_async_copy(rows_v.at[1], acc_sh.at[dst_v.at[0]], ssem).wait()
    plsc.subcore_barrier()
    pltpu.sync_copy(acc_sh.at[nsl], agg_out.at[c, nsl])


_deg_call = pl.kernel(
    _sc_deg,
    out_type=jax.ShapeDtypeStruct((NC, NPAD, F), jnp.float32),
    mesh=_mesh,
    scratch_types=[
        pltpu.VMEM((WROWS, CHUNK), jnp.int32),
        pltpu.VMEM((WROWS, CHUNK), jnp.float32),
        pltpu.VMEM((NSUB,), jnp.float32),
        pltpu.VMEM((NSUB, F), jnp.float32),
        pltpu.SemaphoreType.DMA,
        pltpu.VMEM_SHARED((NPAD,), jnp.float32),
    ],
    compiler_params=_sc_params,
)

_agg_call = pl.kernel(
    _sc_agg,
    out_type=jax.ShapeDtypeStruct((NC, NPAD, F), jnp.float32),
    mesh=_mesh,
    scratch_types=[
        pltpu.VMEM((WROWS, CHUNK), jnp.int32),
        pltpu.VMEM((WROWS, CHUNK), jnp.int32),
        pltpu.VMEM((WROWS, CHUNK), jnp.float32),
        pltpu.VMEM((4, CHUNK, F), jnp.float32),
        pltpu.VMEM((NSUB, F), jnp.float32),
        pltpu.SemaphoreType.DMA,
        pltpu.SemaphoreType.DMA,
        pltpu.VMEM_SHARED((NPAD, F), jnp.float32),
    ],
    compiler_params=_sc_params,
)


GROWS = N * F // 128
GPAD = NPAD * F // 128


def _tc_first(x_ref, w_ref, degp_ref, t_ref, dis_ref):
    dis = lax.rsqrt(degp_ref[0] + degp_ref[1] + 1.0)
    dis_ref[...] = dis
    h0 = jnp.dot(x_ref[...], w_ref[...], preferred_element_type=jnp.float32)
    t_ref[...] = dis[:GROWS] * h0


def _tc_mid(agg_ref, t_ref, dis_ref, b_ref, w_ref, o_ref):
    dis = dis_ref[...][:GROWS]
    h = jax.nn.relu(dis * (agg_ref[0, :GROWS] + agg_ref[1, :GROWS]
                           + t_ref[...]) + b_ref[...])
    o_ref[...] = dis * jnp.dot(h, w_ref[...],
                               preferred_element_type=jnp.float32)


def _tc_final(agg_ref, t_ref, dis_ref, b_ref, o_ref):
    dis = dis_ref[...][:GROWS]
    o_ref[...] = (dis * (agg_ref[0, :GROWS] + agg_ref[1, :GROWS]
                         + t_ref[...]) + b_ref[...])


def _tc(body, out_shape, *args):
    return pl.pallas_call(body, out_shape=out_shape)(*args)


@jax.jit
def kernel(x, edge_index, edge_weight, W1, b1, W2, b2, Wmu, bmu, Wlv, blv):
    f32 = jnp.float32
    src = edge_index[0].astype(jnp.int32)
    dst = edge_index[1].astype(jnp.int32)
    ew = edge_weight.astype(f32)

    npad_e = EPAD - E
    fill = jnp.arange(npad_e, dtype=jnp.int32)
    src_p = jnp.concatenate([src, fill % N]).reshape(ROWS, CHUNK)
    dst_p = jnp.concatenate([dst, N + fill % (NPAD - N)]).reshape(ROWS, CHUNK)
    ew_p = jnp.concatenate([ew, jnp.zeros((npad_e,), f32)]).reshape(ROWS, CHUNK)

    W2p = jnp.pad(W2, ((0, 0), (0, F - W2.shape[1])))
    Wout = jnp.concatenate([Wmu, Wlv], axis=1)
    Woutp = jnp.pad(Wout, ((0, F - Wout.shape[0]), (0, F - Wout.shape[1])))

    def blockdiag(w):
        k = w.shape[0]
        out = jnp.zeros((8 * k, 128), f32)
        for i in range(8):
            out = out.at[i * k:(i + 1) * k, i * F:(i + 1) * F].set(w)
        return out

    W1blk = blockdiag(W1)
    W2blk = blockdiag(W2p)
    Woutblk = blockdiag(Woutp)
    b1r = jnp.tile(b1, 8).reshape(1, 128)
    b2r = jnp.tile(jnp.pad(b2, (0, F - b2.shape[0])), 8).reshape(1, 128)

    xp = x.reshape(GROWS, 8 * x.shape[1])
    degp = _deg_call(dst_p, ew_p).reshape(NC, GPAD, 128)
    t0, dis = _tc(_tc_first,
                  (jax.ShapeDtypeStruct((GROWS, 128), f32),
                   jax.ShapeDtypeStruct((GPAD, 128), f32)),
                  xp, W1blk, degp)

    agg1 = _agg_call(src_p, dst_p, ew_p, t0.reshape(N, F))
    t1 = _tc(_tc_mid, jax.ShapeDtypeStruct((GROWS, 128), f32),
             agg1.reshape(NC, GPAD, 128), t0, dis, b1r, W2blk)
    agg2 = _agg_call(src_p, dst_p, ew_p, t1.reshape(N, F))
    t2 = _tc(_tc_mid, jax.ShapeDtypeStruct((GROWS, 128), f32),
             agg2.reshape(NC, GPAD, 128), t1, dis, b2r, Woutblk)
    agg3 = _agg_call(src_p, dst_p, ew_p, t2.reshape(N, F))
    bout = jnp.tile(jnp.concatenate([bmu, blv, jnp.zeros((F - 4,), f32)]),
                    8).reshape(1, 128)
    out = _tc(_tc_final, jax.ShapeDtypeStruct((GROWS, 128), f32),
              agg3.reshape(NC, GPAD, 128), t2, dis, bout)
    out16 = out.reshape(N, F)
    return (out16[:, 0:2], out16[:, 2:4])


# speedup vs baseline: 1.8287x; 1.1181x over previous
import jax
import jax.numpy as jnp
from jax import lax
from jax.experimental import pallas as pl
from jax.experimental.pallas import tpu as pltpu
from jax.experimental.pallas import tpu_sc as plsc

N = 10000
E = 160000
NPAD = 10240
EPAD = 163840
CHUNK = 256
ROWS = EPAD // CHUNK
NC, NS = 2, 16
NW = NC * NS
WROWS = ROWS // NW
NSUB = NPAD // NS
F = 16

_mesh = plsc.VectorSubcoreMesh(core_axis_name="c", subcore_axis_name="s")
_sc_params = pltpu.CompilerParams(needs_layout_passes=False,
                                  use_tc_tiling_on_sc=False)


def _zero_fill(ref, nrows):
    def body(i, _):
        ref[i] = jnp.zeros((16,), jnp.float32)
        return 0
    lax.fori_loop(0, nrows, body, 0)


def _sc_deg(dst_hbm, ew_hbm, deg_out, idx_v, val_v, z_v, rep_v, sem, deg_sh):
    c = lax.axis_index("c")
    s = lax.axis_index("s")
    w = c * NS + s
    nsl = pl.ds(s * NSUB, NSUB)

    def zbody(k, _):
        z_v[pl.ds(k * 16, 16)] = jnp.zeros((16,), jnp.float32)
        return 0
    lax.fori_loop(0, NSUB // 16, zbody, 0)
    pltpu.sync_copy(z_v, deg_sh.at[nsl])

    wsl = pl.ds(w * WROWS, WROWS)
    pltpu.sync_copy(dst_hbm.at[wsl], idx_v)
    pltpu.sync_copy(ew_hbm.at[wsl], val_v)
    plsc.subcore_barrier()

    def fire(r, _):
        pltpu.async_copy(val_v.at[r], deg_sh.at[idx_v.at[r]], sem, add=True)
        return 0
    lax.fori_loop(0, WROWS, fire, 0)

    def drain(r, _):
        pltpu.make_async_copy(val_v.at[0], deg_sh.at[idx_v.at[0]], sem).wait()
        return 0
    lax.fori_loop(0, WROWS, drain, 0)
    plsc.subcore_barrier()

    pltpu.sync_copy(deg_sh.at[nsl], z_v)

    def rep(k, _):
        dv = z_v[pl.ds(k * 16, 16)]
        for u in range(16):
            rep_v[k * 16 + u] = jnp.full((16,), dv[u], jnp.float32)
        return 0
    lax.fori_loop(0, NSUB // 16, rep, 0)
    pltpu.sync_copy(rep_v, deg_out.at[c, nsl])


def _sc_agg(src_hbm, dst_hbm, ew_hbm, h_hbm, agg_out,
            src_v, dst_v, ew_v, rows_v, zn_v, gsem, ssem, acc_sh):
    c = lax.axis_index("c")
    s = lax.axis_index("s")
    w = c * NS + s
    nsl = pl.ds(s * NSUB, NSUB)

    _zero_fill(zn_v, NSUB)
    pltpu.sync_copy(zn_v, acc_sh.at[nsl])

    wsl = pl.ds(w * WROWS, WROWS)
    pltpu.sync_copy(src_hbm.at[wsl], src_v)
    pltpu.sync_copy(dst_hbm.at[wsl], dst_v)
    pltpu.sync_copy(ew_hbm.at[wsl], ew_v)
    plsc.subcore_barrier()

    pltpu.async_copy(h_hbm.at[src_v.at[0]], rows_v.at[0], gsem)
    pltpu.async_copy(h_hbm.at[src_v.at[1]], rows_v.at[1], gsem)

    def outer(g, _):
        for b in range(4):
            ci = g * 4 + b
            pltpu.make_async_copy(
                h_hbm.at[src_v.at[0]], rows_v.at[b], gsem).wait()

            @pl.when(ci >= 2)
            def _():
                pltpu.make_async_copy(
                    rows_v.at[(b + 2) % 4], acc_sh.at[dst_v.at[0]],
                    ssem).wait()

            @pl.when(ci + 2 < WROWS)
            def _():
                pltpu.async_copy(
                    h_hbm.at[src_v.at[ci + 2]], rows_v.at[(b + 2) % 4], gsem)

            for k in range(CHUNK // 16):
                base = k * 16
                ewv = ew_v[ci, pl.ds(base, 16)]
                for u in range(16):
                    j = base + u
                    rows_v[b, j] = rows_v[b, j] * ewv[u]

            pltpu.async_copy(rows_v.at[b], acc_sh.at[dst_v.at[ci]], ssem,
                             add=True)
        return 0
    lax.fori_loop(0, WROWS // 4, outer, 0)
    pltpu.make_async_copy(rows_v.at[0], acc_sh.at[dst_v.at[0]], ssem).wait()
    pltpu.make_async_copy(rows_v.at[1], acc_sh.at[dst_v.at[0]], ssem).wait()
    plsc.subcore_barrier()
    pltpu.sync_copy(acc_sh.at[nsl], agg_out.at[c, nsl])


_deg_call = pl.kernel(
    _sc_deg,
    out_type=jax.ShapeDtypeStruct((NC, NPAD, F), jnp.float32),
    mesh=_mesh,
    scratch_types=[
        pltpu.VMEM((WROWS, CHUNK), jnp.int32),
        pltpu.VMEM((WROWS, CHUNK), jnp.float32),
        pltpu.VMEM((NSUB,), jnp.float32),
        pltpu.VMEM((NSUB, F), jnp.float32),
        pltpu.SemaphoreType.DMA,
        pltpu.VMEM_SHARED((NPAD,), jnp.float32),
    ],
    compiler_params=_sc_params,
)

_agg_call = pl.kernel(
    _sc_agg,
    out_type=jax.ShapeDtypeStruct((NC, NPAD, F), jnp.float32),
    mesh=_mesh,
    scratch_types=[
        pltpu.VMEM((WROWS, CHUNK), jnp.int32),
        pltpu.VMEM((WROWS, CHUNK), jnp.int32),
        pltpu.VMEM((WROWS, CHUNK), jnp.float32),
        pltpu.VMEM((4, CHUNK, F), jnp.float32),
        pltpu.VMEM((NSUB, F), jnp.float32),
        pltpu.SemaphoreType.DMA,
        pltpu.SemaphoreType.DMA,
        pltpu.VMEM_SHARED((NPAD, F), jnp.float32),
    ],
    compiler_params=_sc_params,
)


GROWS = N * F // 128
GPAD = NPAD * F // 128


def _tc_first(x_ref, w_ref, degp_ref, t_ref, dis_ref):
    dis = lax.rsqrt(degp_ref[0] + degp_ref[1] + 1.0)
    dis_ref[...] = dis
    parts = [jnp.dot(x_ref[:, r, :], w_ref[...],
                     preferred_element_type=jnp.float32) for r in range(8)]
    t_ref[...] = dis[:GROWS] * jnp.concatenate(parts, axis=1)


def _tc_mid(agg_ref, t_ref, dis_ref, b_ref, w_ref, o_ref):
    dis = dis_ref[...][:GROWS]
    h = jax.nn.relu(dis * (agg_ref[0, :GROWS] + agg_ref[1, :GROWS]
                           + t_ref[...]) + b_ref[...])
    parts = [jnp.dot(h[:, r * F:(r + 1) * F], w_ref[...],
                     preferred_element_type=jnp.float32) for r in range(8)]
    o_ref[...] = dis * jnp.concatenate(parts, axis=1)


def _tc_final(agg_ref, t_ref, dis_ref, b_ref, o_ref):
    dis = dis_ref[...][:GROWS]
    o_ref[...] = (dis * (agg_ref[0, :GROWS] + agg_ref[1, :GROWS]
                         + t_ref[...]) + b_ref[...])


def _tc(body, out_shape, *args):
    return pl.pallas_call(body, out_shape=out_shape)(*args)


@jax.jit
def kernel(x, edge_index, edge_weight, W1, b1, W2, b2, Wmu, bmu, Wlv, blv):
    f32 = jnp.float32
    src = edge_index[0].astype(jnp.int32)
    dst = edge_index[1].astype(jnp.int32)
    ew = edge_weight.astype(f32)

    npad_e = EPAD - E
    fill = jnp.arange(npad_e, dtype=jnp.int32)
    src_p = jnp.concatenate([src, fill % N]).reshape(ROWS, CHUNK)
    dst_p = jnp.concatenate([dst, N + fill % (NPAD - N)]).reshape(ROWS, CHUNK)
    ew_p = jnp.concatenate([ew, jnp.zeros((npad_e,), f32)]).reshape(ROWS, CHUNK)

    W2p = jnp.pad(W2, ((0, 0), (0, F - W2.shape[1])))
    Wout = jnp.concatenate([Wmu, Wlv], axis=1)
    Woutp = jnp.pad(Wout, ((0, F - Wout.shape[0]), (0, F - Wout.shape[1])))

    b1r = jnp.tile(b1, 8).reshape(1, 128)
    b2r = jnp.tile(jnp.pad(b2, (0, F - b2.shape[0])), 8).reshape(1, 128)

    x8 = x.reshape(GROWS, 8, x.shape[1])
    degp = _deg_call(dst_p, ew_p).reshape(NC, GPAD, 128)
    t0, dis = _tc(_tc_first,
                  (jax.ShapeDtypeStruct((GROWS, 128), f32),
                   jax.ShapeDtypeStruct((GPAD, 128), f32)),
                  x8, W1, degp)

    agg1 = _agg_call(src_p, dst_p, ew_p, t0.reshape(N, F))
    t1 = _tc(_tc_mid, jax.ShapeDtypeStruct((GROWS, 128), f32),
             agg1.reshape(NC, GPAD, 128), t0, dis, b1r, W2p)
    agg2 = _agg_call(src_p, dst_p, ew_p, t1.reshape(N, F))
    t2 = _tc(_tc_mid, jax.ShapeDtypeStruct((GROWS, 128), f32),
             agg2.reshape(NC, GPAD, 128), t1, dis, b2r, Woutp)
    agg3 = _agg_call(src_p, dst_p, ew_p, t2.reshape(N, F))
    bout = jnp.tile(jnp.concatenate([bmu, blv, jnp.zeros((F - 4,), f32)]),
                    8).reshape(1, 128)
    out = _tc(_tc_final, jax.ShapeDtypeStruct((GROWS, 128), f32),
              agg3.reshape(NC, GPAD, 128), t2, dis, bout)
    out16 = out.reshape(N, F)
    return (out16[:, 0:2], out16[:, 2:4])


# speedup vs baseline: 2.0924x; 1.1442x over previous
import jax
import jax.numpy as jnp
from jax import lax
from jax.experimental import pallas as pl
from jax.experimental.pallas import tpu as pltpu
from jax.experimental.pallas import tpu_sc as plsc

N = 10000
E = 160000
NPAD = 10240
EPAD = 163840
CHUNK = 256
ROWS = EPAD // CHUNK
NC, NS = 2, 16
NW = NC * NS
WROWS = ROWS // NW
NSUB = NPAD // NS
F = 16

_mesh = plsc.VectorSubcoreMesh(core_axis_name="c", subcore_axis_name="s")
_sc_params = pltpu.CompilerParams(needs_layout_passes=False,
                                  use_tc_tiling_on_sc=False)


def _zero_fill(ref, nrows):
    def body(i, _):
        ref[i] = jnp.zeros((16,), jnp.float32)
        return 0
    lax.fori_loop(0, nrows, body, 0)


def _sc_deg(dst_hbm, ew_hbm, deg_out, idx_v, val_v, z_v, rep_v, sem, deg_sh):
    c = lax.axis_index("c")
    s = lax.axis_index("s")
    w = c * NS + s
    nsl = pl.ds(s * NSUB, NSUB)

    def zbody(k, _):
        z_v[pl.ds(k * 16, 16)] = jnp.zeros((16,), jnp.float32)
        return 0
    lax.fori_loop(0, NSUB // 16, zbody, 0)
    pltpu.sync_copy(z_v, deg_sh.at[nsl])

    wsl = pl.ds(w * WROWS, WROWS)
    pltpu.sync_copy(dst_hbm.at[wsl], idx_v)
    pltpu.sync_copy(ew_hbm.at[wsl], val_v)
    plsc.subcore_barrier()

    def fire(r, _):
        pltpu.async_copy(val_v.at[r], deg_sh.at[idx_v.at[r]], sem, add=True)
        return 0
    lax.fori_loop(0, WROWS, fire, 0)

    def drain(r, _):
        pltpu.make_async_copy(val_v.at[0], deg_sh.at[idx_v.at[0]], sem).wait()
        return 0
    lax.fori_loop(0, WROWS, drain, 0)
    plsc.subcore_barrier()

    pltpu.sync_copy(deg_sh.at[nsl], z_v)

    def rep(k, _):
        dv = z_v[pl.ds(k * 16, 16)]
        for u in range(16):
            rep_v[k * 16 + u] = jnp.full((16,), dv[u], jnp.float32)
        return 0
    lax.fori_loop(0, NSUB // 16, rep, 0)
    pltpu.sync_copy(rep_v, deg_out.at[c, nsl])


def _sc_agg(src_hbm, dst_hbm, ew_hbm, h_hbm, agg_out,
            src_v, dst_v, ew_v, rows_v, zn_v, gsem, ssem, acc_sh, h_sh):
    c = lax.axis_index("c")
    s = lax.axis_index("s")
    w = c * NS + s
    nsl = pl.ds(s * NSUB, NSUB)

    _zero_fill(zn_v, NSUB)
    pltpu.sync_copy(zn_v, acc_sh.at[nsl])

    wsl = pl.ds(w * WROWS, WROWS)
    pltpu.sync_copy(src_hbm.at[wsl], src_v)
    pltpu.sync_copy(dst_hbm.at[wsl], dst_v)
    pltpu.sync_copy(ew_hbm.at[wsl], ew_v)

    @pl.when(s < NS - 1)
    def _():
        pltpu.sync_copy(h_hbm.at[pl.ds(s * 640, 640)],
                        h_sh.at[pl.ds(s * 640, 640)])

    @pl.when(s == NS - 1)
    def _():
        pltpu.sync_copy(h_hbm.at[pl.ds(9600, 400)], h_sh.at[pl.ds(9600, 400)])
    plsc.subcore_barrier()

    pltpu.async_copy(h_sh.at[src_v.at[0]], rows_v.at[0], gsem)
    pltpu.async_copy(h_sh.at[src_v.at[1]], rows_v.at[1], gsem)

    def outer(g, _):
        for b in range(4):
            ci = g * 4 + b
            pltpu.make_async_copy(
                h_sh.at[src_v.at[0]], rows_v.at[b], gsem).wait()

            @pl.when(ci >= 2)
            def _():
                pltpu.make_async_copy(
                    rows_v.at[(b + 2) % 4], acc_sh.at[dst_v.at[0]],
                    ssem).wait()

            @pl.when(ci + 2 < WROWS)
            def _():
                pltpu.async_copy(
                    h_sh.at[src_v.at[ci + 2]], rows_v.at[(b + 2) % 4], gsem)

            for k in range(CHUNK // 16):
                base = k * 16
                ewv = ew_v[ci, pl.ds(base, 16)]
                for u in range(16):
                    j = base + u
                    rows_v[b, j] = rows_v[b, j] * ewv[u]

            pltpu.async_copy(rows_v.at[b], acc_sh.at[dst_v.at[ci]], ssem,
                             add=True)
        return 0
    lax.fori_loop(0, WROWS // 4, outer, 0)
    pltpu.make_async_copy(rows_v.at[0], acc_sh.at[dst_v.at[0]], ssem).wait()
    pltpu.make_async_copy(rows_v.at[1], acc_sh.at[dst_v.at[0]], ssem).wait()
    plsc.subcore_barrier()
    pltpu.sync_copy(acc_sh.at[nsl], agg_out.at[c, nsl])


_deg_call = pl.kernel(
    _sc_deg,
    out_type=jax.ShapeDtypeStruct((NC, NPAD, F), jnp.float32),
    mesh=_mesh,
    scratch_types=[
        pltpu.VMEM((WROWS, CHUNK), jnp.int32),
        pltpu.VMEM((WROWS, CHUNK), jnp.float32),
        pltpu.VMEM((NSUB,), jnp.float32),
        pltpu.VMEM((NSUB, F), jnp.float32),
        pltpu.SemaphoreType.DMA,
        pltpu.VMEM_SHARED((NPAD,), jnp.float32),
    ],
    compiler_params=_sc_params,
)

_agg_call = pl.kernel(
    _sc_agg,
    out_type=jax.ShapeDtypeStruct((NC, NPAD, F), jnp.float32),
    mesh=_mesh,
    scratch_types=[
        pltpu.VMEM((WROWS, CHUNK), jnp.int32),
        pltpu.VMEM((WROWS, CHUNK), jnp.int32),
        pltpu.VMEM((WROWS, CHUNK), jnp.float32),
        pltpu.VMEM((4, CHUNK, F), jnp.float32),
        pltpu.VMEM((NSUB, F), jnp.float32),
        pltpu.SemaphoreType.DMA,
        pltpu.SemaphoreType.DMA,
        pltpu.VMEM_SHARED((NPAD, F), jnp.float32),
        pltpu.VMEM_SHARED((N, F), jnp.float32),
    ],
    compiler_params=_sc_params,
)


GROWS = N * F // 128
GPAD = NPAD * F // 128


def _tc_first(x_ref, w_ref, degp_ref, t_ref, dis_ref):
    dis = lax.rsqrt(degp_ref[0] + degp_ref[1] + 1.0)
    dis_ref[...] = dis
    parts = [jnp.dot(x_ref[:, r, :], w_ref[...],
                     preferred_element_type=jnp.float32) for r in range(8)]
    t_ref[...] = dis[:GROWS] * jnp.concatenate(parts, axis=1)


def _tc_mid(agg_ref, t_ref, dis_ref, b_ref, w_ref, o_ref):
    dis = dis_ref[...][:GROWS]
    h = jax.nn.relu(dis * (agg_ref[0, :GROWS] + agg_ref[1, :GROWS]
                           + t_ref[...]) + b_ref[...])
    parts = [jnp.dot(h[:, r * F:(r + 1) * F], w_ref[...],
                     preferred_element_type=jnp.float32) for r in range(8)]
    o_ref[...] = dis * jnp.concatenate(parts, axis=1)


def _tc_final(agg_ref, t_ref, dis_ref, b_ref, o_ref):
    dis = dis_ref[...][:GROWS]
    o_ref[...] = (dis * (agg_ref[0, :GROWS] + agg_ref[1, :GROWS]
                         + t_ref[...]) + b_ref[...])


def _tc(body, out_shape, *args):
    return pl.pallas_call(body, out_shape=out_shape)(*args)


@jax.jit
def kernel(x, edge_index, edge_weight, W1, b1, W2, b2, Wmu, bmu, Wlv, blv):
    f32 = jnp.float32
    src = edge_index[0].astype(jnp.int32)
    dst = edge_index[1].astype(jnp.int32)
    ew = edge_weight.astype(f32)

    npad_e = EPAD - E
    fill = jnp.arange(npad_e, dtype=jnp.int32)
    src_p = jnp.concatenate([src, fill % N]).reshape(ROWS, CHUNK)
    dst_p = jnp.concatenate([dst, N + fill % (NPAD - N)]).reshape(ROWS, CHUNK)
    ew_p = jnp.concatenate([ew, jnp.zeros((npad_e,), f32)]).reshape(ROWS, CHUNK)

    W2p = jnp.pad(W2, ((0, 0), (0, F - W2.shape[1])))
    Wout = jnp.concatenate([Wmu, Wlv], axis=1)
    Woutp = jnp.pad(Wout, ((0, F - Wout.shape[0]), (0, F - Wout.shape[1])))

    b1r = jnp.tile(b1, 8).reshape(1, 128)
    b2r = jnp.tile(jnp.pad(b2, (0, F - b2.shape[0])), 8).reshape(1, 128)

    x8 = x.reshape(GROWS, 8, x.shape[1])
    degp = _deg_call(dst_p, ew_p).reshape(NC, GPAD, 128)
    t0, dis = _tc(_tc_first,
                  (jax.ShapeDtypeStruct((GROWS, 128), f32),
                   jax.ShapeDtypeStruct((GPAD, 128), f32)),
                  x8, W1, degp)

    agg1 = _agg_call(src_p, dst_p, ew_p, t0.reshape(N, F))
    t1 = _tc(_tc_mid, jax.ShapeDtypeStruct((GROWS, 128), f32),
             agg1.reshape(NC, GPAD, 128), t0, dis, b1r, W2p)
    agg2 = _agg_call(src_p, dst_p, ew_p, t1.reshape(N, F))
    t2 = _tc(_tc_mid, jax.ShapeDtypeStruct((GROWS, 128), f32),
             agg2.reshape(NC, GPAD, 128), t1, dis, b2r, Woutp)
    agg3 = _agg_call(src_p, dst_p, ew_p, t2.reshape(N, F))
    bout = jnp.tile(jnp.concatenate([bmu, blv, jnp.zeros((F - 4,), f32)]),
                    8).reshape(1, 128)
    out = _tc(_tc_final, jax.ShapeDtypeStruct((GROWS, 128), f32),
              agg3.reshape(NC, GPAD, 128), t2, dis, bout)
    out16 = out.reshape(N, F)
    return (out16[:, 0:2], out16[:, 2:4])
